# Initial kernel scaffold; baseline (speedup 1.0000x reference)
import jax
import jax.numpy as jnp
from jax.experimental import pallas as pl


def kernel(status, est_size, degree, edges, edge_ids, edge_pos, emb_table, att_w1, att_b1, att_w2, att_b2, conv1_w, conv1_b, conv2_w, conv2_b, actor_w1, actor_b1, actor_w2, actor_b2, critic_w1, critic_b1, critic_w2, critic_b2, critic_w3, critic_b3):
    raise NotImplementedError("write your pallas kernel here")


# speedup vs baseline: 4.0305x; 4.0305x over previous
import jax
import jax.numpy as jnp
from jax.experimental import pallas as pl
from jax.experimental.pallas import tpu as pltpu

N = 10000
E = 320000
HID = 256


def _edge_kernel(edges_ref, eid_ref, pos_ref, w1_ref, b1_ref, w2_ref, b2_ref,
                 v_ref, src_ref, dst_ref):
    eid = eid_ref[...]
    pos = pos_ref[...]
    oh = (pos == jax.lax.broadcasted_iota(jnp.int32, (pos.shape[0], 3), 1)
          ).astype(jnp.float32)
    feats = jnp.concatenate([eid, oh], axis=1)
    hid = jnp.maximum(
        jnp.dot(feats, w1_ref[...], preferred_element_type=jnp.float32)
        + b1_ref[...], 0.0)
    s = (jnp.dot(hid, w2_ref[...], preferred_element_type=jnp.float32)
         + b2_ref[...])
    expw = jnp.exp(s)
    ones = jnp.ones_like(expw)
    zeros = jnp.zeros((pos.shape[0], 7), jnp.float32)
    v_ref[...] = jnp.concatenate([feats * expw, expw, ones, zeros], axis=1)
    src_ref[...] = edges_ref[:, 0:1]
    dst_ref[...] = edges_ref[:, 1:2]


def _node1_kernel(sc_ref, acc_ref, nf_ref, dis_ref):
    n = sc_ref.shape[0]
    acc = acc_ref[...]
    agg = acc[:, :7] / jnp.maximum(acc[:, 7:8], 1e-38)
    deg = acc[:, 8:9] + 1.0
    dis = jax.lax.rsqrt(deg)

    def norm(col):
        m = jnp.mean(col)
        var = jnp.sum((col - m) ** 2) / (n - 1)
        std = jnp.sqrt(var)
        safe = jnp.where(std > 1e-8, std, 1.0)
        return jnp.where(std > 1e-8, (col - m) / safe, col - m)

    nf_ref[...] = jnp.concatenate(
        [sc_ref[:, 0:1], norm(sc_ref[:, 1:2]), norm(sc_ref[:, 2:3]), agg,
         jnp.zeros((n, 6), jnp.float32)], axis=1)
    dis_ref[...] = dis


def _mm1_kernel(nf_ref, dis_ref, w_ref, h1p_ref):
    h1 = jnp.dot(nf_ref[...], w_ref[...], preferred_element_type=jnp.float32)
    h1p = h1 * dis_ref[...]
    h1p_ref[0] = h1p[:, :128]
    h1p_ref[1] = h1p[:, 128:]


def _mm2_kernel(acc1_ref, h1p_ref, dis_ref, b1_ref, w2_ref, h2p_ref):
    dis = dis_ref[...]
    agg1a = dis * (acc1_ref[0] + h1p_ref[0])
    agg1b = dis * (acc1_ref[1] + h1p_ref[1])
    x1 = jnp.maximum(
        jnp.concatenate([agg1a, agg1b], axis=1) + b1_ref[...], 0.0)
    h2 = jnp.dot(x1, w2_ref[...], preferred_element_type=jnp.float32)
    h2p = h2 * dis
    h2p_ref[0] = h2p[:, :128]
    h2p_ref[1] = h2p[:, 128:]


def _x2_kernel(acc2_ref, h2p_ref, dis_ref, b2_ref, x2_ref, psum_ref):
    dis = dis_ref[...]
    agg2a = dis * (acc2_ref[0] + h2p_ref[0])
    agg2b = dis * (acc2_ref[1] + h2p_ref[1])
    x2 = jnp.maximum(
        jnp.concatenate([agg2a, agg2b], axis=1) + b2_ref[...], 0.0)
    x2_ref[...] = x2
    psum_ref[...] = jnp.sum(x2, axis=0, keepdims=True)[None]


def _heads_kernel(x2_ref, psum_ref,
                  aw1_ref, ab1_ref, aw2_ref, ab2_ref,
                  cw1_ref, cb1_ref, cw2_ref, cb2_ref, cw3_ref, cb3_ref,
                  logits_ref, value_ref):
    x2 = x2_ref[...]
    g = jnp.sum(psum_ref[:, 0, :], axis=0, keepdims=True) * (1.0 / N)
    comb = jnp.concatenate(
        [x2, jnp.broadcast_to(g, (x2.shape[0], HID))], axis=1)
    ha = jnp.maximum(
        jnp.dot(comb, aw1_ref[...], preferred_element_type=jnp.float32)
        + ab1_ref[...], 0.0)
    logits_ref[...] = (jnp.dot(ha, aw2_ref[...],
                               preferred_element_type=jnp.float32)
                       + ab2_ref[...])

    @pl.when(pl.program_id(0) == 0)
    def _():
        pooled = jnp.concatenate([g, g], axis=1)
        h1 = jnp.maximum(
            jnp.dot(pooled, cw1_ref[...], preferred_element_type=jnp.float32)
            + cb1_ref[...], 0.0)
        h2 = jnp.maximum(
            jnp.dot(h1, cw2_ref[...], preferred_element_type=jnp.float32)
            + cb2_ref[...], 0.0)
        value_ref[...] = (jnp.sum(h2 * cw3_ref[...].T, axis=1,
                                  keepdims=True) + cb3_ref[...])


def kernel(status, est_size, degree, edges, edge_ids, edge_pos, emb_table,
           att_w1, att_b1, att_w2, att_b2, conv1_w, conv1_b, conv2_w, conv2_b,
           actor_w1, actor_b1, actor_w2, actor_b2,
           critic_w1, critic_b1, critic_w2, critic_b2, critic_w3, critic_b3):
    f32 = jnp.float32
    edges = edges.astype(jnp.int32)

    eid = emb_table[edge_ids]

    EB = 2000
    egrid = E // EB
    v, src, dst = pl.pallas_call(
        _edge_kernel,
        grid=(egrid,),
        in_specs=[
            pl.BlockSpec((EB, 2), lambda i: (i, 0)),
            pl.BlockSpec((EB, 4), lambda i: (i, 0)),
            pl.BlockSpec((EB, 1), lambda i: (i, 0)),
            pl.BlockSpec((7, 16), lambda i: (0, 0)),
            pl.BlockSpec((1, 16), lambda i: (0, 0)),
            pl.BlockSpec((16, 1), lambda i: (0, 0)),
            pl.BlockSpec((1, 1), lambda i: (0, 0)),
        ],
        out_specs=[
            pl.BlockSpec((EB, 16), lambda i: (i, 0)),
            pl.BlockSpec((EB, 1), lambda i: (i, 0)),
            pl.BlockSpec((EB, 1), lambda i: (i, 0)),
        ],
        out_shape=[
            jax.ShapeDtypeStruct((E, 16), f32),
            jax.ShapeDtypeStruct((E, 1), jnp.int32),
            jax.ShapeDtypeStruct((E, 1), jnp.int32),
        ],
    )(edges, eid, edge_pos.astype(jnp.int32).reshape(E, 1), att_w1,
      att_b1.reshape(1, 16), att_w2, att_b2.reshape(1, 1))
    src = src.reshape(E)
    dst = dst.reshape(E)

    accD = jnp.zeros((N, 16), f32).at[dst].add(v)
    accS = jnp.zeros((N, 16), f32).at[src].add(v)
    acc = accD + accS
    acc = acc.at[:, 8].set(accD[:, 8])

    sc = jnp.stack([status, est_size, degree, jnp.zeros((N,), f32)], axis=1)
    nf, dis = pl.pallas_call(
        _node1_kernel,
        out_shape=[jax.ShapeDtypeStruct((N, 16), f32),
                   jax.ShapeDtypeStruct((N, 1), f32)],
    )(sc, acc)

    NB = 2000
    ngrid = N // NB

    w1pad = jnp.concatenate([conv1_w, jnp.zeros((6, HID), f32)], axis=0)
    h1p = pl.pallas_call(
        _mm1_kernel,
        grid=(ngrid,),
        in_specs=[
            pl.BlockSpec((NB, 16), lambda i: (i, 0)),
            pl.BlockSpec((NB, 1), lambda i: (i, 0)),
            pl.BlockSpec((16, HID), lambda i: (0, 0)),
        ],
        out_specs=pl.BlockSpec((2, NB, 128), lambda i: (0, i, 0)),
        out_shape=jax.ShapeDtypeStruct((2, N, 128), f32),
    )(nf, dis, w1pad)

    h1p_full = jnp.concatenate([h1p[0], h1p[1]], axis=1)
    acc1 = jnp.zeros((N, HID), f32).at[dst].add(h1p_full[src])
    acc1 = acc1.reshape(N, 2, 128).transpose(1, 0, 2)

    h2p = pl.pallas_call(
        _mm2_kernel,
        grid=(ngrid,),
        in_specs=[
            pl.BlockSpec((2, NB, 128), lambda i: (0, i, 0)),
            pl.BlockSpec((2, NB, 128), lambda i: (0, i, 0)),
            pl.BlockSpec((NB, 1), lambda i: (i, 0)),
            pl.BlockSpec((1, HID), lambda i: (0, 0)),
            pl.BlockSpec((HID, HID), lambda i: (0, 0)),
        ],
        out_specs=pl.BlockSpec((2, NB, 128), lambda i: (0, i, 0)),
        out_shape=jax.ShapeDtypeStruct((2, N, 128), f32),
    )(acc1, h1p, dis, conv1_b.reshape(1, HID), conv2_w)

    h2p_full = jnp.concatenate([h2p[0], h2p[1]], axis=1)
    acc2 = jnp.zeros((N, HID), f32).at[dst].add(h2p_full[src])
    acc2 = acc2.reshape(N, 2, 128).transpose(1, 0, 2)

    x2, psum = pl.pallas_call(
        _x2_kernel,
        grid=(ngrid,),
        in_specs=[
            pl.BlockSpec((2, NB, 128), lambda i: (0, i, 0)),
            pl.BlockSpec((2, NB, 128), lambda i: (0, i, 0)),
            pl.BlockSpec((NB, 1), lambda i: (i, 0)),
            pl.BlockSpec((1, HID), lambda i: (0, 0)),
        ],
        out_specs=[pl.BlockSpec((NB, HID), lambda i: (i, 0)),
                   pl.BlockSpec((1, 1, HID), lambda i: (i, 0, 0))],
        out_shape=[jax.ShapeDtypeStruct((N, HID), f32),
                   jax.ShapeDtypeStruct((ngrid, 1, HID), f32)],
    )(acc2, h2p, dis, conv2_b.reshape(1, HID))

    logits, value = pl.pallas_call(
        _heads_kernel,
        grid=(ngrid,),
        in_specs=[
            pl.BlockSpec((NB, HID), lambda i: (i, 0)),
            pl.BlockSpec((ngrid, 1, HID), lambda i: (0, 0, 0)),
            pl.BlockSpec((2 * HID, HID), lambda i: (0, 0)),
            pl.BlockSpec((1, HID), lambda i: (0, 0)),
            pl.BlockSpec((HID, 1), lambda i: (0, 0)),
            pl.BlockSpec((1, 1), lambda i: (0, 0)),
            pl.BlockSpec((2 * HID, 2 * HID), lambda i: (0, 0)),
            pl.BlockSpec((1, 2 * HID), lambda i: (0, 0)),
            pl.BlockSpec((2 * HID, HID), lambda i: (0, 0)),
            pl.BlockSpec((1, HID), lambda i: (0, 0)),
            pl.BlockSpec((HID, 1), lambda i: (0, 0)),
            pl.BlockSpec((1, 1), lambda i: (0, 0)),
        ],
        out_specs=[pl.BlockSpec((NB, 1), lambda i: (i, 0)),
                   pl.BlockSpec((1, 1), lambda i: (0, 0))],
        out_shape=[jax.ShapeDtypeStruct((N, 1), f32),
                   jax.ShapeDtypeStruct((1, 1), f32)],
    )(x2, psum,
      actor_w1, actor_b1.reshape(1, HID), actor_w2, actor_b2.reshape(1, 1),
      critic_w1, critic_b1.reshape(1, 2 * HID), critic_w2,
      critic_b2.reshape(1, HID), critic_w3, critic_b3.reshape(1, 1))
    return (logits.reshape(N), value[0, 0])


# speedup vs baseline: 7.3160x; 1.8152x over previous
import jax
import jax.numpy as jnp
from jax import lax
from jax.experimental import pallas as pl
from jax.experimental.pallas import tpu as pltpu
from jax.experimental.pallas import tpu_sc as plsc

N = 10000
E = 320000
HID = 256

NTILES = 16
ETILE = E // NTILES
ECH = 125
NCH = ETILE // ECH
NROWS = 624
SLABCH = 16
NREM = N - NTILES * NROWS


def _gcn_agg_body(table_ref, src_ref, dst_ref, zeros_ref, out_ref,
                  src_slab, dst_slab, buf, acc):
    c = lax.axis_index("c")
    s = lax.axis_index("s")
    pltpu.sync_copy(zeros_ref, acc.at[pl.ds(s * NROWS, NROWS)])

    @pl.when(s == 0)
    def _():
        pltpu.sync_copy(zeros_ref.at[pl.ds(0, NREM)],
                        acc.at[pl.ds(NTILES * NROWS, NREM)])

    plsc.subcore_barrier()
    tbl = table_ref.at[c]
    def outer(o, carry):
        pltpu.sync_copy(src_ref.at[s, pl.ds(o * SLABCH, SLABCH)], src_slab)
        pltpu.sync_copy(dst_ref.at[s, pl.ds(o * SLABCH, SLABCH)], dst_slab)

        def body(j, carry2):
            pltpu.sync_copy(tbl.at[src_slab.at[j]], buf)
            pltpu.sync_copy(buf, acc.at[dst_slab.at[j]], add=True)
            return carry2

        lax.fori_loop(0, SLABCH, body, 0, unroll=False)
        return carry

    lax.fori_loop(0, NCH // SLABCH, outer, 0, unroll=False)
    plsc.subcore_barrier()
    pltpu.sync_copy(acc.at[pl.ds(s * NROWS, NROWS)],
                    out_ref.at[c, pl.ds(s * NROWS, NROWS)])

    @pl.when(s == 0)
    def _():
        pltpu.sync_copy(acc.at[pl.ds(NTILES * NROWS, NREM)],
                        out_ref.at[c, pl.ds(NTILES * NROWS, NREM)])


def _gcn_agg(table, src3, dst3, zeros):
    mesh = plsc.VectorSubcoreMesh(core_axis_name="c", subcore_axis_name="s")
    return pl.kernel(
        _gcn_agg_body,
        out_type=jax.ShapeDtypeStruct((2, N, 128), jnp.float32),
        mesh=mesh,
        scratch_types=[
            pltpu.VMEM((SLABCH, ECH), jnp.int32),
            pltpu.VMEM((SLABCH, ECH), jnp.int32),
            pltpu.VMEM((ECH, 128), jnp.float32),
            pltpu.VMEM_SHARED((N, 128), jnp.float32),
        ],
    )(table, src3, dst3, zeros)


def _edge_kernel(edges_ref, eid_ref, pos_ref, w1_ref, b1_ref, w2_ref, b2_ref,
                 v_ref, src_ref, dst_ref):
    eid = eid_ref[...]
    pos = pos_ref[...]
    oh = (pos == jax.lax.broadcasted_iota(jnp.int32, (pos.shape[0], 3), 1)
          ).astype(jnp.float32)
    feats = jnp.concatenate([eid, oh], axis=1)
    hid = jnp.maximum(
        jnp.dot(feats, w1_ref[...], preferred_element_type=jnp.float32)
        + b1_ref[...], 0.0)
    s = (jnp.dot(hid, w2_ref[...], preferred_element_type=jnp.float32)
         + b2_ref[...])
    expw = jnp.exp(s)
    ones = jnp.ones_like(expw)
    zeros = jnp.zeros((pos.shape[0], 7), jnp.float32)
    v_ref[...] = jnp.concatenate([feats * expw, expw, ones, zeros], axis=1)
    src_ref[...] = edges_ref[:, 0:1]
    dst_ref[...] = edges_ref[:, 1:2]


def _node1_kernel(sc_ref, acc_ref, nf_ref, dis_ref):
    n = sc_ref.shape[0]
    acc = acc_ref[...]
    agg = acc[:, :7] / jnp.maximum(acc[:, 7:8], 1e-38)
    deg = acc[:, 8:9] + 1.0
    dis = jax.lax.rsqrt(deg)

    def norm(col):
        m = jnp.mean(col)
        var = jnp.sum((col - m) ** 2) / (n - 1)
        std = jnp.sqrt(var)
        safe = jnp.where(std > 1e-8, std, 1.0)
        return jnp.where(std > 1e-8, (col - m) / safe, col - m)

    nf_ref[...] = jnp.concatenate(
        [sc_ref[:, 0:1], norm(sc_ref[:, 1:2]), norm(sc_ref[:, 2:3]), agg,
         jnp.zeros((n, 6), jnp.float32)], axis=1)
    dis_ref[...] = dis


def _mm1_kernel(nf_ref, dis_ref, w_ref, h1p_ref):
    h1 = jnp.dot(nf_ref[...], w_ref[...], preferred_element_type=jnp.float32)
    h1p = h1 * dis_ref[...]
    h1p_ref[0] = h1p[:, :128]
    h1p_ref[1] = h1p[:, 128:]


def _mm2_kernel(acc1_ref, h1p_ref, dis_ref, b1_ref, w2_ref, h2p_ref):
    dis = dis_ref[...]
    agg1a = dis * (acc1_ref[0] + h1p_ref[0])
    agg1b = dis * (acc1_ref[1] + h1p_ref[1])
    x1 = jnp.maximum(
        jnp.concatenate([agg1a, agg1b], axis=1) + b1_ref[...], 0.0)
    h2 = jnp.dot(x1, w2_ref[...], preferred_element_type=jnp.float32)
    h2p = h2 * dis
    h2p_ref[0] = h2p[:, :128]
    h2p_ref[1] = h2p[:, 128:]


def _x2_kernel(acc2_ref, h2p_ref, dis_ref, b2_ref, x2_ref, psum_ref):
    dis = dis_ref[...]
    agg2a = dis * (acc2_ref[0] + h2p_ref[0])
    agg2b = dis * (acc2_ref[1] + h2p_ref[1])
    x2 = jnp.maximum(
        jnp.concatenate([agg2a, agg2b], axis=1) + b2_ref[...], 0.0)
    x2_ref[...] = x2
    psum_ref[...] = jnp.sum(x2, axis=0, keepdims=True)[None]


def _heads_kernel(x2_ref, psum_ref,
                  aw1_ref, ab1_ref, aw2_ref, ab2_ref,
                  cw1_ref, cb1_ref, cw2_ref, cb2_ref, cw3_ref, cb3_ref,
                  logits_ref, value_ref):
    x2 = x2_ref[...]
    g = jnp.sum(psum_ref[:, 0, :], axis=0, keepdims=True) * (1.0 / N)
    comb = jnp.concatenate(
        [x2, jnp.broadcast_to(g, (x2.shape[0], HID))], axis=1)
    ha = jnp.maximum(
        jnp.dot(comb, aw1_ref[...], preferred_element_type=jnp.float32)
        + ab1_ref[...], 0.0)
    logits_ref[...] = (jnp.dot(ha, aw2_ref[...],
                               preferred_element_type=jnp.float32)
                       + ab2_ref[...])

    @pl.when(pl.program_id(0) == 0)
    def _():
        pooled = jnp.concatenate([g, g], axis=1)
        h1 = jnp.maximum(
            jnp.dot(pooled, cw1_ref[...], preferred_element_type=jnp.float32)
            + cb1_ref[...], 0.0)
        h2 = jnp.maximum(
            jnp.dot(h1, cw2_ref[...], preferred_element_type=jnp.float32)
            + cb2_ref[...], 0.0)
        value_ref[...] = (jnp.sum(h2 * cw3_ref[...].T, axis=1,
                                  keepdims=True) + cb3_ref[...])


def kernel(status, est_size, degree, edges, edge_ids, edge_pos, emb_table,
           att_w1, att_b1, att_w2, att_b2, conv1_w, conv1_b, conv2_w, conv2_b,
           actor_w1, actor_b1, actor_w2, actor_b2,
           critic_w1, critic_b1, critic_w2, critic_b2, critic_w3, critic_b3):
    f32 = jnp.float32
    edges = edges.astype(jnp.int32)

    eid = emb_table[edge_ids]

    EB = 2000
    egrid = E // EB
    v, src, dst = pl.pallas_call(
        _edge_kernel,
        grid=(egrid,),
        in_specs=[
            pl.BlockSpec((EB, 2), lambda i: (i, 0)),
            pl.BlockSpec((EB, 4), lambda i: (i, 0)),
            pl.BlockSpec((EB, 1), lambda i: (i, 0)),
            pl.BlockSpec((7, 16), lambda i: (0, 0)),
            pl.BlockSpec((1, 16), lambda i: (0, 0)),
            pl.BlockSpec((16, 1), lambda i: (0, 0)),
            pl.BlockSpec((1, 1), lambda i: (0, 0)),
        ],
        out_specs=[
            pl.BlockSpec((EB, 16), lambda i: (i, 0)),
            pl.BlockSpec((EB, 1), lambda i: (i, 0)),
            pl.BlockSpec((EB, 1), lambda i: (i, 0)),
        ],
        out_shape=[
            jax.ShapeDtypeStruct((E, 16), f32),
            jax.ShapeDtypeStruct((E, 1), jnp.int32),
            jax.ShapeDtypeStruct((E, 1), jnp.int32),
        ],
    )(edges, eid, edge_pos.astype(jnp.int32).reshape(E, 1), att_w1,
      att_b1.reshape(1, 16), att_w2, att_b2.reshape(1, 1))
    src = src.reshape(E)
    dst = dst.reshape(E)

    accD = jnp.zeros((N, 16), f32).at[dst].add(v)
    accS = jnp.zeros((N, 16), f32).at[src].add(v)
    acc = accD + accS
    acc = acc.at[:, 8].set(accD[:, 8])

    sc = jnp.stack([status, est_size, degree, jnp.zeros((N,), f32)], axis=1)
    nf, dis = pl.pallas_call(
        _node1_kernel,
        out_shape=[jax.ShapeDtypeStruct((N, 16), f32),
                   jax.ShapeDtypeStruct((N, 1), f32)],
    )(sc, acc)

    NB = 2000
    ngrid = N // NB

    w1pad = jnp.concatenate([conv1_w, jnp.zeros((6, HID), f32)], axis=0)
    h1p = pl.pallas_call(
        _mm1_kernel,
        grid=(ngrid,),
        in_specs=[
            pl.BlockSpec((NB, 16), lambda i: (i, 0)),
            pl.BlockSpec((NB, 1), lambda i: (i, 0)),
            pl.BlockSpec((16, HID), lambda i: (0, 0)),
        ],
        out_specs=pl.BlockSpec((2, NB, 128), lambda i: (0, i, 0)),
        out_shape=jax.ShapeDtypeStruct((2, N, 128), f32),
    )(nf, dis, w1pad)

    src3 = src.reshape(NTILES, NCH, ECH)
    dst3 = dst.reshape(NTILES, NCH, ECH)
    zrows = jnp.zeros((NROWS, 128), f32)
    acc1 = _gcn_agg(h1p, src3, dst3, zrows)

    h2p = pl.pallas_call(
        _mm2_kernel,
        grid=(ngrid,),
        in_specs=[
            pl.BlockSpec((2, NB, 128), lambda i: (0, i, 0)),
            pl.BlockSpec((2, NB, 128), lambda i: (0, i, 0)),
            pl.BlockSpec((NB, 1), lambda i: (i, 0)),
            pl.BlockSpec((1, HID), lambda i: (0, 0)),
            pl.BlockSpec((HID, HID), lambda i: (0, 0)),
        ],
        out_specs=pl.BlockSpec((2, NB, 128), lambda i: (0, i, 0)),
        out_shape=jax.ShapeDtypeStruct((2, N, 128), f32),
    )(acc1, h1p, dis, conv1_b.reshape(1, HID), conv2_w)

    acc2 = _gcn_agg(h2p, src3, dst3, zrows)

    x2, psum = pl.pallas_call(
        _x2_kernel,
        grid=(ngrid,),
        in_specs=[
            pl.BlockSpec((2, NB, 128), lambda i: (0, i, 0)),
            pl.BlockSpec((2, NB, 128), lambda i: (0, i, 0)),
            pl.BlockSpec((NB, 1), lambda i: (i, 0)),
            pl.BlockSpec((1, HID), lambda i: (0, 0)),
        ],
        out_specs=[pl.BlockSpec((NB, HID), lambda i: (i, 0)),
                   pl.BlockSpec((1, 1, HID), lambda i: (i, 0, 0))],
        out_shape=[jax.ShapeDtypeStruct((N, HID), f32),
                   jax.ShapeDtypeStruct((ngrid, 1, HID), f32)],
    )(acc2, h2p, dis, conv2_b.reshape(1, HID))

    logits, value = pl.pallas_call(
        _heads_kernel,
        grid=(ngrid,),
        in_specs=[
            pl.BlockSpec((NB, HID), lambda i: (i, 0)),
            pl.BlockSpec((ngrid, 1, HID), lambda i: (0, 0, 0)),
            pl.BlockSpec((2 * HID, HID), lambda i: (0, 0)),
            pl.BlockSpec((1, HID), lambda i: (0, 0)),
            pl.BlockSpec((HID, 1), lambda i: (0, 0)),
            pl.BlockSpec((1, 1), lambda i: (0, 0)),
            pl.BlockSpec((2 * HID, 2 * HID), lambda i: (0, 0)),
            pl.BlockSpec((1, 2 * HID), lambda i: (0, 0)),
            pl.BlockSpec((2 * HID, HID), lambda i: (0, 0)),
            pl.BlockSpec((1, HID), lambda i: (0, 0)),
            pl.BlockSpec((HID, 1), lambda i: (0, 0)),
            pl.BlockSpec((1, 1), lambda i: (0, 0)),
        ],
        out_specs=[pl.BlockSpec((NB, 1), lambda i: (i, 0)),
                   pl.BlockSpec((1, 1), lambda i: (0, 0))],
        out_shape=[jax.ShapeDtypeStruct((N, 1), f32),
                   jax.ShapeDtypeStruct((1, 1), f32)],
    )(x2, psum,
      actor_w1, actor_b1.reshape(1, HID), actor_w2, actor_b2.reshape(1, 1),
      critic_w1, critic_b1.reshape(1, 2 * HID), critic_w2,
      critic_b2.reshape(1, HID), critic_w3, critic_b3.reshape(1, 1))
    return (logits.reshape(N), value[0, 0])


# speedup vs baseline: 11.2675x; 1.5401x over previous
import jax
import jax.numpy as jnp
from jax import lax
from jax.experimental import pallas as pl
from jax.experimental.pallas import tpu as pltpu
from jax.experimental.pallas import tpu_sc as plsc

N = 10000
E = 320000
HID = 256

NTILES = 16
ETILE = E // NTILES
ECH = 125
NCH = ETILE // ECH
NROWS = 624
SLABCH = 16
NREM = N - NTILES * NROWS


def _gcn_agg_body(table_ref, src_ref, dst_ref, zeros_ref, out_ref,
                  src_slab, dst_slab, buf, acc):
    c = lax.axis_index("c")
    s = lax.axis_index("s")
    pltpu.sync_copy(zeros_ref, acc.at[pl.ds(s * NROWS, NROWS)])

    @pl.when(s == 0)
    def _():
        pltpu.sync_copy(zeros_ref.at[pl.ds(0, NREM)],
                        acc.at[pl.ds(NTILES * NROWS, NREM)])

    plsc.subcore_barrier()
    tbl = table_ref.at[c]
    def outer(o, carry):
        pltpu.sync_copy(src_ref.at[s, pl.ds(o * SLABCH, SLABCH)], src_slab)
        pltpu.sync_copy(dst_ref.at[s, pl.ds(o * SLABCH, SLABCH)], dst_slab)

        def body(j, carry2):
            pltpu.sync_copy(tbl.at[src_slab.at[j]], buf)
            pltpu.sync_copy(buf, acc.at[dst_slab.at[j]], add=True)
            return carry2

        lax.fori_loop(0, SLABCH, body, 0, unroll=False)
        return carry

    lax.fori_loop(0, NCH // SLABCH, outer, 0, unroll=False)
    plsc.subcore_barrier()
    pltpu.sync_copy(acc.at[pl.ds(s * NROWS, NROWS)],
                    out_ref.at[c, pl.ds(s * NROWS, NROWS)])

    @pl.when(s == 0)
    def _():
        pltpu.sync_copy(acc.at[pl.ds(NTILES * NROWS, NREM)],
                        out_ref.at[c, pl.ds(NTILES * NROWS, NREM)])


def _gcn_agg(table, src3, dst3, zeros):
    mesh = plsc.VectorSubcoreMesh(core_axis_name="c", subcore_axis_name="s")
    return pl.kernel(
        _gcn_agg_body,
        out_type=jax.ShapeDtypeStruct((2, N, 128), jnp.float32),
        mesh=mesh,
        scratch_types=[
            pltpu.VMEM((SLABCH, ECH), jnp.int32),
            pltpu.VMEM((SLABCH, ECH), jnp.int32),
            pltpu.VMEM((ECH, 128), jnp.float32),
            pltpu.VMEM_SHARED((N, 128), jnp.float32),
        ],
    )(table, src3, dst3, zeros)


ACH = 125
ANCH = (E // 2 // NTILES) // ACH


def _att_scatter_body(v_ref, idx_ref, zeros_ref, out_ref,
                      idx_slab, buf, acc):
    c = lax.axis_index("c")
    s = lax.axis_index("s")
    pltpu.sync_copy(zeros_ref, acc.at[pl.ds(s * NROWS, NROWS)])

    @pl.when(s == 0)
    def _():
        pltpu.sync_copy(zeros_ref.at[pl.ds(0, NREM)],
                        acc.at[pl.ds(NTILES * NROWS, NREM)])

    plsc.subcore_barrier()

    def outer(o, carry):
        pltpu.sync_copy(idx_ref.at[c, s, pl.ds(o * SLABCH, SLABCH)], idx_slab)

        def body(j, carry2):
            pltpu.sync_copy(v_ref.at[c, s, o * SLABCH + j], buf)
            pltpu.sync_copy(buf, acc.at[idx_slab.at[j]], add=True)
            return carry2

        lax.fori_loop(0, SLABCH, body, 0, unroll=False)
        return carry

    lax.fori_loop(0, ANCH // SLABCH, outer, 0, unroll=False)
    plsc.subcore_barrier()
    pltpu.sync_copy(acc.at[pl.ds(s * NROWS, NROWS)],
                    out_ref.at[c, pl.ds(s * NROWS, NROWS)])

    @pl.when(s == 0)
    def _():
        pltpu.sync_copy(acc.at[pl.ds(NTILES * NROWS, NREM)],
                        out_ref.at[c, pl.ds(NTILES * NROWS, NREM)])


def _att_scatter(v5, idx4, zeros16):
    mesh = plsc.VectorSubcoreMesh(core_axis_name="c", subcore_axis_name="s")
    return pl.kernel(
        _att_scatter_body,
        out_type=jax.ShapeDtypeStruct((2, N, 16), jnp.float32),
        mesh=mesh,
        scratch_types=[
            pltpu.VMEM((SLABCH, ACH), jnp.int32),
            pltpu.VMEM((ACH, 16), jnp.float32),
            pltpu.VMEM_SHARED((N, 16), jnp.float32),
        ],
    )(v5, idx4, zeros16)


def _edge_kernel(edges_ref, eid_ref, pos_ref, w1_ref, b1_ref, w2_ref, b2_ref,
                 v_ref, src_ref, dst_ref):
    eid = eid_ref[...]
    pos = pos_ref[...]
    oh = (pos == jax.lax.broadcasted_iota(jnp.int32, (pos.shape[0], 3), 1)
          ).astype(jnp.float32)
    feats = jnp.concatenate([eid, oh], axis=1)
    hid = jnp.maximum(
        jnp.dot(feats, w1_ref[...], preferred_element_type=jnp.float32)
        + b1_ref[...], 0.0)
    s = (jnp.dot(hid, w2_ref[...], preferred_element_type=jnp.float32)
         + b2_ref[...])
    expw = jnp.exp(s)
    ones = jnp.ones_like(expw)
    zeros = jnp.zeros((pos.shape[0], 7), jnp.float32)
    v_ref[...] = jnp.concatenate([feats * expw, expw, ones, zeros], axis=1)
    src_ref[...] = edges_ref[:, 0:1]
    dst_ref[...] = edges_ref[:, 1:2]


def _node1_kernel(sc_ref, attD_ref, attS_ref, nf_ref, dis_ref):
    n = sc_ref.shape[0]
    accD = attD_ref[0] + attD_ref[1]
    acc = accD + attS_ref[0] + attS_ref[1]
    agg = acc[:, :7] / jnp.maximum(acc[:, 7:8], 1e-38)
    deg = accD[:, 8:9] + 1.0
    dis = jax.lax.rsqrt(deg)

    def norm(col):
        m = jnp.mean(col)
        var = jnp.sum((col - m) ** 2) / (n - 1)
        std = jnp.sqrt(var)
        safe = jnp.where(std > 1e-8, std, 1.0)
        return jnp.where(std > 1e-8, (col - m) / safe, col - m)

    nf_ref[...] = jnp.concatenate(
        [sc_ref[:, 0:1], norm(sc_ref[:, 1:2]), norm(sc_ref[:, 2:3]), agg,
         jnp.zeros((n, 6), jnp.float32)], axis=1)
    dis_ref[...] = dis


def _mm1_kernel(nf_ref, dis_ref, w_ref, h1p_ref):
    h1 = jnp.dot(nf_ref[...], w_ref[...], preferred_element_type=jnp.float32)
    h1p = h1 * dis_ref[...]
    h1p_ref[0] = h1p[:, :128]
    h1p_ref[1] = h1p[:, 128:]


def _mm2_kernel(acc1_ref, h1p_ref, dis_ref, b1_ref, w2_ref, h2p_ref):
    dis = dis_ref[...]
    agg1a = dis * (acc1_ref[0] + h1p_ref[0])
    agg1b = dis * (acc1_ref[1] + h1p_ref[1])
    x1 = jnp.maximum(
        jnp.concatenate([agg1a, agg1b], axis=1) + b1_ref[...], 0.0)
    h2 = jnp.dot(x1, w2_ref[...], preferred_element_type=jnp.float32)
    h2p = h2 * dis
    h2p_ref[0] = h2p[:, :128]
    h2p_ref[1] = h2p[:, 128:]


def _x2_kernel(acc2_ref, h2p_ref, dis_ref, b2_ref, x2_ref, psum_ref):
    dis = dis_ref[...]
    agg2a = dis * (acc2_ref[0] + h2p_ref[0])
    agg2b = dis * (acc2_ref[1] + h2p_ref[1])
    x2 = jnp.maximum(
        jnp.concatenate([agg2a, agg2b], axis=1) + b2_ref[...], 0.0)
    x2_ref[...] = x2
    psum_ref[...] = jnp.sum(x2, axis=0, keepdims=True)[None]


def _heads_kernel(x2_ref, psum_ref,
                  aw1_ref, ab1_ref, aw2_ref, ab2_ref,
                  cw1_ref, cb1_ref, cw2_ref, cb2_ref, cw3_ref, cb3_ref,
                  logits_ref, value_ref):
    x2 = x2_ref[...]
    g = jnp.sum(psum_ref[:, 0, :], axis=0, keepdims=True) * (1.0 / N)
    comb = jnp.concatenate(
        [x2, jnp.broadcast_to(g, (x2.shape[0], HID))], axis=1)
    ha = jnp.maximum(
        jnp.dot(comb, aw1_ref[...], preferred_element_type=jnp.float32)
        + ab1_ref[...], 0.0)
    logits_ref[...] = (jnp.dot(ha, aw2_ref[...],
                               preferred_element_type=jnp.float32)
                       + ab2_ref[...])

    @pl.when(pl.program_id(0) == 0)
    def _():
        pooled = jnp.concatenate([g, g], axis=1)
        h1 = jnp.maximum(
            jnp.dot(pooled, cw1_ref[...], preferred_element_type=jnp.float32)
            + cb1_ref[...], 0.0)
        h2 = jnp.maximum(
            jnp.dot(h1, cw2_ref[...], preferred_element_type=jnp.float32)
            + cb2_ref[...], 0.0)
        value_ref[...] = (jnp.sum(h2 * cw3_ref[...].T, axis=1,
                                  keepdims=True) + cb3_ref[...])


def kernel(status, est_size, degree, edges, edge_ids, edge_pos, emb_table,
           att_w1, att_b1, att_w2, att_b2, conv1_w, conv1_b, conv2_w, conv2_b,
           actor_w1, actor_b1, actor_w2, actor_b2,
           critic_w1, critic_b1, critic_w2, critic_b2, critic_w3, critic_b3):
    f32 = jnp.float32
    edges = edges.astype(jnp.int32)

    eid = emb_table[edge_ids]

    EB = 2000
    egrid = E // EB
    v, src, dst = pl.pallas_call(
        _edge_kernel,
        grid=(egrid,),
        in_specs=[
            pl.BlockSpec((EB, 2), lambda i: (i, 0)),
            pl.BlockSpec((EB, 4), lambda i: (i, 0)),
            pl.BlockSpec((EB, 1), lambda i: (i, 0)),
            pl.BlockSpec((7, 16), lambda i: (0, 0)),
            pl.BlockSpec((1, 16), lambda i: (0, 0)),
            pl.BlockSpec((16, 1), lambda i: (0, 0)),
            pl.BlockSpec((1, 1), lambda i: (0, 0)),
        ],
        out_specs=[
            pl.BlockSpec((EB, 16), lambda i: (i, 0)),
            pl.BlockSpec((EB, 1), lambda i: (i, 0)),
            pl.BlockSpec((EB, 1), lambda i: (i, 0)),
        ],
        out_shape=[
            jax.ShapeDtypeStruct((E, 16), f32),
            jax.ShapeDtypeStruct((E, 1), jnp.int32),
            jax.ShapeDtypeStruct((E, 1), jnp.int32),
        ],
    )(edges, eid, edge_pos.astype(jnp.int32).reshape(E, 1), att_w1,
      att_b1.reshape(1, 16), att_w2, att_b2.reshape(1, 1))
    src = src.reshape(E)
    dst = dst.reshape(E)

    v5 = v.reshape(2, NTILES, ANCH, ACH, 16)
    src4 = src.reshape(2, NTILES, ANCH, ACH)
    dst4 = dst.reshape(2, NTILES, ANCH, ACH)
    zeros16 = jnp.zeros((NROWS, 16), f32)
    attD = _att_scatter(v5, dst4, zeros16)
    attS = _att_scatter(v5, src4, zeros16)

    sc = jnp.stack([status, est_size, degree, jnp.zeros((N,), f32)], axis=1)
    nf, dis = pl.pallas_call(
        _node1_kernel,
        out_shape=[jax.ShapeDtypeStruct((N, 16), f32),
                   jax.ShapeDtypeStruct((N, 1), f32)],
    )(sc, attD, attS)

    NB = 2000
    ngrid = N // NB

    w1pad = jnp.concatenate([conv1_w, jnp.zeros((6, HID), f32)], axis=0)
    h1p = pl.pallas_call(
        _mm1_kernel,
        grid=(ngrid,),
        in_specs=[
            pl.BlockSpec((NB, 16), lambda i: (i, 0)),
            pl.BlockSpec((NB, 1), lambda i: (i, 0)),
            pl.BlockSpec((16, HID), lambda i: (0, 0)),
        ],
        out_specs=pl.BlockSpec((2, NB, 128), lambda i: (0, i, 0)),
        out_shape=jax.ShapeDtypeStruct((2, N, 128), f32),
    )(nf, dis, w1pad)

    src3 = src.reshape(NTILES, NCH, ECH)
    dst3 = dst.reshape(NTILES, NCH, ECH)
    zrows = jnp.zeros((NROWS, 128), f32)
    acc1 = _gcn_agg(h1p, src3, dst3, zrows)

    h2p = pl.pallas_call(
        _mm2_kernel,
        grid=(ngrid,),
        in_specs=[
            pl.BlockSpec((2, NB, 128), lambda i: (0, i, 0)),
            pl.BlockSpec((2, NB, 128), lambda i: (0, i, 0)),
            pl.BlockSpec((NB, 1), lambda i: (i, 0)),
            pl.BlockSpec((1, HID), lambda i: (0, 0)),
            pl.BlockSpec((HID, HID), lambda i: (0, 0)),
        ],
        out_specs=pl.BlockSpec((2, NB, 128), lambda i: (0, i, 0)),
        out_shape=jax.ShapeDtypeStruct((2, N, 128), f32),
    )(acc1, h1p, dis, conv1_b.reshape(1, HID), conv2_w)

    acc2 = _gcn_agg(h2p, src3, dst3, zrows)

    x2, psum = pl.pallas_call(
        _x2_kernel,
        grid=(ngrid,),
        in_specs=[
            pl.BlockSpec((2, NB, 128), lambda i: (0, i, 0)),
            pl.BlockSpec((2, NB, 128), lambda i: (0, i, 0)),
            pl.BlockSpec((NB, 1), lambda i: (i, 0)),
            pl.BlockSpec((1, HID), lambda i: (0, 0)),
        ],
        out_specs=[pl.BlockSpec((NB, HID), lambda i: (i, 0)),
                   pl.BlockSpec((1, 1, HID), lambda i: (i, 0, 0))],
        out_shape=[jax.ShapeDtypeStruct((N, HID), f32),
                   jax.ShapeDtypeStruct((ngrid, 1, HID), f32)],
    )(acc2, h2p, dis, conv2_b.reshape(1, HID))

    logits, value = pl.pallas_call(
        _heads_kernel,
        grid=(ngrid,),
        in_specs=[
            pl.BlockSpec((NB, HID), lambda i: (i, 0)),
            pl.BlockSpec((ngrid, 1, HID), lambda i: (0, 0, 0)),
            pl.BlockSpec((2 * HID, HID), lambda i: (0, 0)),
            pl.BlockSpec((1, HID), lambda i: (0, 0)),
            pl.BlockSpec((HID, 1), lambda i: (0, 0)),
            pl.BlockSpec((1, 1), lambda i: (0, 0)),
            pl.BlockSpec((2 * HID, 2 * HID), lambda i: (0, 0)),
            pl.BlockSpec((1, 2 * HID), lambda i: (0, 0)),
            pl.BlockSpec((2 * HID, HID), lambda i: (0, 0)),
            pl.BlockSpec((1, HID), lambda i: (0, 0)),
            pl.BlockSpec((HID, 1), lambda i: (0, 0)),
            pl.BlockSpec((1, 1), lambda i: (0, 0)),
        ],
        out_specs=[pl.BlockSpec((NB, 1), lambda i: (i, 0)),
                   pl.BlockSpec((1, 1), lambda i: (0, 0))],
        out_shape=[jax.ShapeDtypeStruct((N, 1), f32),
                   jax.ShapeDtypeStruct((1, 1), f32)],
    )(x2, psum,
      actor_w1, actor_b1.reshape(1, HID), actor_w2, actor_b2.reshape(1, 1),
      critic_w1, critic_b1.reshape(1, 2 * HID), critic_w2,
      critic_b2.reshape(1, HID), critic_w3, critic_b3.reshape(1, 1))
    return (logits.reshape(N), value[0, 0])
